# baseline (speedup 1.0000x reference)
import functools

import jax
import jax.numpy as jnp
from jax import lax
from jax.experimental import pallas as pl
from jax.experimental.pallas import tpu as pltpu
from jax.experimental.pallas import tpu_sc as plsc

_NUM_CORES = 2
_NUM_SUBCORES = 16
_NW = _NUM_CORES * _NUM_SUBCORES
_LANES = 16


def _splat(v):
  return jnp.full((_LANES,), v, jnp.int32)


@functools.partial(jax.jit, static_argnums=(0,))
def _pos_embed(dims, row_w, col_w, hei_w):
  B, Z, C, X, Y = dims
  K = X * Y
  CH = col_w.shape[1]
  CH2 = 2 * CH
  RPW = C // _NW
  CHUNK = RPW * K
  DMA_WINDOW = 32

  mesh = plsc.VectorSubcoreMesh(
      core_axis_name="c", subcore_axis_name="s",
      num_cores=_NUM_CORES, num_subcores=_NUM_SUBCORES)

  @functools.partial(
      pl.kernel,
      out_type=jax.ShapeDtypeStruct((B * Z * C * K,), jnp.float32),
      mesh=mesh,
      scratch_types=[
          pltpu.VMEM((row_w.size,), jnp.float32),
          pltpu.VMEM((col_w.size,), jnp.float32),
          pltpu.VMEM((hei_w.size,), jnp.float32),
          pltpu.VMEM((Z * CHUNK,), jnp.float32),
          pltpu.SemaphoreType.DMA,
      ],
      compiler_params=pltpu.CompilerParams(needs_layout_passes=False),
  )
  def kern(row_hbm, col_hbm, hei_hbm, out_hbm, roww_v, colw_v, heiw_v,
           slabs_v, sem):
    cid = lax.axis_index("c")
    sid = lax.axis_index("s")
    wid = sid * _NUM_CORES + cid
    c0 = wid * RPW
    zdep = c0 + RPW > CH2
    nslab = jnp.where(zdep, Z, 1)

    pltpu.sync_copy(row_hbm, roww_v)
    pltpu.sync_copy(col_hbm, colw_v)
    pltpu.sync_copy(hei_hbm, heiw_v)

    idx16 = lax.iota(jnp.int32, 16)

    def build_row(z, r):
      c = c0 + r
      off = (z * RPW + r) * K

      def col_case():
        ga = plsc.load_gather(colw_v, [idx16 * CH + c])
        gb = plsc.load_gather(colw_v, [(idx16 + 16) * CH + c])

        def st(m, _):
          slabs_v[pl.ds(off + 32 * m, 16)] = ga
          slabs_v[pl.ds(off + 32 * m + 16, 16)] = gb
          return 0
        lax.fori_loop(0, X, st, 0)

      def row_case():
        def st(x, _):
          s = plsc.load_gather(roww_v, [_splat(x * CH + c - CH)])
          slabs_v[pl.ds(off + 32 * x, 16)] = s
          slabs_v[pl.ds(off + 32 * x + 16, 16)] = s
          return 0
        lax.fori_loop(0, X, st, 0)

      def hei_case():
        s = plsc.load_gather(heiw_v, [_splat(z * CH + c - CH2)])

        def st(m, _):
          slabs_v[pl.ds(off + 16 * m, 16)] = s
          return 0
        lax.fori_loop(0, K // 16, st, 0)

      lax.cond(c < CH, col_case,
               lambda: lax.cond(c < CH2, row_case, hei_case))

    def build_slab(z, _):
      def row_body(r, _):
        build_row(z, r)
        return 0
      lax.fori_loop(0, RPW, row_body, 0)
      return 0

    lax.fori_loop(0, nslab, build_slab, 0)

    def wait_one():
      pltpu.make_async_copy(
          slabs_v.at[pl.ds(0, CHUNK)],
          out_hbm.at[pl.ds(c0 * K, CHUNK)], sem).wait()

    def issue(t, _):
      b = t // Z
      z = t - b * Z
      zs = jnp.where(zdep, z, 0)
      pltpu.async_copy(
          slabs_v.at[pl.ds(zs * CHUNK, CHUNK)],
          out_hbm.at[pl.ds((b * Z + z) * (C * K) + c0 * K, CHUNK)], sem)

      @pl.when(t >= DMA_WINDOW)
      def _():
        wait_one()
      return 0

    lax.fori_loop(0, B * Z, issue, 0)

    def drain(t, _):
      wait_one()
      return 0

    lax.fori_loop(0, min(DMA_WINDOW, B * Z), drain, 0)

  out = kern(row_w.reshape(-1), col_w.reshape(-1), hei_w.reshape(-1))
  return out.reshape(B, Z, C, X, Y)


def kernel(tensor, row_w, col_w, hei_w):
  B, Z, C, X, Y = tensor.shape
  assert C % _NW == 0 and X == 32 and Y == 32
  return _pos_embed((B, Z, C, X, Y), row_w, col_w, hei_w)
